# Initial kernel scaffold; baseline (speedup 1.0000x reference)
import jax
import jax.numpy as jnp
from jax.experimental import pallas as pl


def kernel(inputs, targets):
    raise NotImplementedError("write your pallas kernel here")


# speedup vs baseline: 29.6703x; 29.6703x over previous
import functools

import jax
import jax.numpy as jnp
from jax import lax
from jax.experimental import pallas as pl
from jax.experimental.pallas import tpu as pltpu
from jax.experimental.pallas import tpu_sc as plsc

IGNORE = 19
CE_W = 0.5
LV_W = 0.5
NCLS = 10
C = 20
NB = 16384
HSIZE = NCLS * 2 * NB

BATCH, HDIM, WDIM = 4, 512, 512
NPIX = BATCH * HDIM * WDIM
HBLK = 8
GRID_H = HDIM // HBLK

LANES = 128
IDX_ROWS = NCLS * NPIX // LANES
NSC, NTEC = 2, 16
NW = NSC * NTEC
ROWS_PER_W = IDX_ROWS // NW
ROWS_PER_STEP = 16
STEPS = ROWS_PER_W // ROWS_PER_STEP


def _p1_body(x_ref, t_ref, idx_ref, ce_ref):
    first = jnp.logical_and(pl.program_id(0) == 0, pl.program_id(1) == 0)

    @pl.when(first)
    def _():
        ce_ref[...] = jnp.zeros_like(ce_ref)

    x = x_ref[0]
    t = t_ref[0]
    m = jnp.max(x, axis=0)
    ex = jnp.exp(x - m[None])
    s = jnp.sum(ex, axis=0)
    lse = jnp.log(s) + m
    cls = lax.broadcasted_iota(jnp.int32, (C, HBLK, WDIM), 0)
    xt = jnp.sum(jnp.where(t[None] == cls, x, 0.0), axis=0)
    maskf = (t != IGNORE).astype(jnp.float32)
    nll = (lse - xt) * maskf
    ce_ref[0] += jnp.sum(nll.reshape(HBLK, WDIM // 128, 128), axis=1)
    ce_ref[1] += jnp.sum(maskf.reshape(HBLK, WDIM // 128, 128), axis=1)

    for c in range(NCLS):
        p = ex[c] / s
        fg = t == c
        e = jnp.abs(fg.astype(jnp.float32) - p) * maskf
        b = jnp.minimum((e * NB).astype(jnp.int32), NB - 1)
        idx_ref[c, 0] = (c * 2 * NB) + jnp.where(fg, NB, 0) + b


def _p3_body(hist_ref, ce_ref, out_ref):
    hh = hist_ref[0] + hist_ref[1]
    cnt = hh[:, 0, :] + hh[:, 1, :]
    fgc = hh[:, 1, :]

    def cum(v):
        sh = 1
        while sh < NB:
            z = jnp.zeros((NCLS, sh), jnp.float32)
            v = v + jnp.concatenate([z, v[:, :-sh]], axis=1)
            sh *= 2
        return v

    cum_c = cum(cnt)
    cum_f = cum(fgc)
    tot_c = cum_c[:, -1:]
    tot_f = cum_f[:, -1:]
    n_b = tot_c - cum_c + cnt
    f_b = tot_f - cum_f + fgc
    gts = tot_f
    j_end = 1.0 - (gts - f_b) / jnp.maximum(gts + n_b - f_b, 1.0)
    j_sta = 1.0 - (gts - (f_b - fgc)) / jnp.maximum(
        gts + (n_b - cnt) - (f_b - fgc), 1.0)
    eb = (lax.broadcasted_iota(jnp.int32, (NCLS, NB), 1).astype(jnp.float32)
          + 0.5) / NB
    term = jnp.sum(eb * (j_end - j_sta), axis=1, keepdims=True)
    lv = jnp.sum(jnp.where(gts > 0, term, 0.0)) / NCLS
    ce = jnp.sum(ce_ref[0]) / jnp.sum(ce_ref[1])
    out_ref[...] = jnp.full((8, 128), CE_W * ce + LV_W * lv, jnp.float32)


def _sc_hist(idx_hbm, zeros_hbm, out_hbm, idx_v, ones_v, hist_sh):
    cid = lax.axis_index("c")
    sid = lax.axis_index("s")
    wid = sid * NSC + cid

    for i in range(LANES // 16):
        ones_v[pl.ds(16 * i, 16)] = jnp.ones((16,), jnp.float32)

    @pl.when(sid == 0)
    def _():
        pltpu.sync_copy(zeros_hbm, hist_sh)

    plsc.subcore_barrier()

    def step(i, carry):
        row = wid * ROWS_PER_W + i * ROWS_PER_STEP
        pltpu.sync_copy(idx_hbm.at[pl.ds(row, ROWS_PER_STEP)], idx_v)
        for j in range(ROWS_PER_STEP):
            pltpu.sync_copy(ones_v, hist_sh.at[idx_v.at[j]], add=True)
        return carry

    lax.fori_loop(0, STEPS, step, 0)

    plsc.subcore_barrier()

    @pl.when(sid == 0)
    def _():
        pltpu.sync_copy(hist_sh, out_hbm.at[cid])


def kernel(inputs, targets):
    targets = targets.astype(jnp.int32)

    idx, ce_parts = pl.pallas_call(
        _p1_body,
        grid=(BATCH, GRID_H),
        in_specs=[
            pl.BlockSpec((1, C, HBLK, WDIM), lambda b, h: (b, 0, h, 0)),
            pl.BlockSpec((1, HBLK, WDIM), lambda b, h: (b, h, 0)),
        ],
        out_specs=[
            pl.BlockSpec((NCLS, 1, HBLK, WDIM), lambda b, h: (0, b, h, 0)),
            pl.BlockSpec((2, 8, 128), lambda b, h: (0, 0, 0)),
        ],
        out_shape=[
            jax.ShapeDtypeStruct((NCLS, BATCH, HDIM, WDIM), jnp.int32),
            jax.ShapeDtypeStruct((2, 8, 128), jnp.float32),
        ],
    )(inputs, targets)

    idx2d = idx.reshape(IDX_ROWS, LANES)
    zeros = jnp.zeros((HSIZE,), jnp.float32)

    sc_fn = pl.kernel(
        _sc_hist,
        out_type=jax.ShapeDtypeStruct((NSC, HSIZE), jnp.float32),
        mesh=plsc.VectorSubcoreMesh(core_axis_name="c", subcore_axis_name="s"),
        scratch_types=[
            pltpu.VMEM((ROWS_PER_STEP, LANES), jnp.int32),
            pltpu.VMEM((LANES,), jnp.float32),
            pltpu.VMEM_SHARED((HSIZE,), jnp.float32),
        ],
    )
    hist = sc_fn(idx2d, zeros)

    out = pl.pallas_call(
        _p3_body,
        in_specs=[
            pl.BlockSpec((NSC, NCLS, 2, NB), lambda: (0, 0, 0, 0)),
            pl.BlockSpec((2, 8, 128), lambda: (0, 0, 0)),
        ],
        out_specs=pl.BlockSpec((8, 128), lambda: (0, 0)),
        out_shape=jax.ShapeDtypeStruct((8, 128), jnp.float32),
    )(hist.reshape(NSC, NCLS, 2, NB), ce_parts)

    return out[0, 0]


# speedup vs baseline: 49.1121x; 1.6553x over previous
import jax
import jax.numpy as jnp
from jax import lax
from jax.experimental import pallas as pl
from jax.experimental.pallas import tpu as pltpu
from jax.experimental.pallas import tpu_sc as plsc

IGNORE = 19
CE_W = 0.5
LV_W = 0.5
NCLS = 10
C = 20
NB = 2048
HSIZE = 2 * NCLS * NB

BATCH, HDIM, WDIM = 4, 512, 512
NPIX = BATCH * HDIM * WDIM
HBLK = 8
GRID_H = HDIM // HBLK

NIDX = NCLS * NPIX
NSC, NTEC = 2, 16
NW = NSC * NTEC
PER_W = NIDX // NW
CHUNK = 32768
NCHUNK = PER_W // CHUNK
GROUPS = CHUNK // 2048


def _p1_body(x_ref, t_ref, idx_ref, ce_ref):
    first = jnp.logical_and(pl.program_id(0) == 0, pl.program_id(1) == 0)

    @pl.when(first)
    def _():
        ce_ref[...] = jnp.zeros_like(ce_ref)

    x = x_ref[0]
    t = t_ref[0]
    m = jnp.max(x, axis=0)
    ex = jnp.exp(x - m[None])
    s = jnp.sum(ex, axis=0)
    lse = jnp.log(s) + m
    cls = lax.broadcasted_iota(jnp.int32, (C, HBLK, WDIM), 0)
    xt = jnp.sum(jnp.where(t[None] == cls, x, 0.0), axis=0)
    maskf = (t != IGNORE).astype(jnp.float32)
    nll = (lse - xt) * maskf
    ce_ref[0] += jnp.sum(nll.reshape(HBLK, WDIM // 128, 128), axis=1)
    ce_ref[1] += jnp.sum(maskf.reshape(HBLK, WDIM // 128, 128), axis=1)

    for c in range(NCLS):
        p = ex[c] / s
        fg = t == c
        e = jnp.abs(fg.astype(jnp.float32) - p) * maskf
        b = jnp.minimum((e * NB).astype(jnp.int32), NB - 1)
        idx_ref[c, 0] = c * NB + b + jnp.where(fg, NCLS * NB, 0)


def _p3_body(hist_ref, ce_ref, out_ref):
    tot = hist_ref[pl.ds(0, C)]
    for w in range(1, NW):
        tot += hist_ref[pl.ds(w * C, C)]
    fgc = tot[NCLS:]
    cnt = tot[:NCLS] + fgc

    def cum(v):
        sh = 1
        while sh < NB:
            z = jnp.zeros((NCLS, sh), jnp.float32)
            v = v + jnp.concatenate([z, v[:, :-sh]], axis=1)
            sh *= 2
        return v

    cum_c = cum(cnt)
    cum_f = cum(fgc)
    tot_c = cum_c[:, -1:]
    tot_f = cum_f[:, -1:]
    n_b = tot_c - cum_c + cnt
    f_b = tot_f - cum_f + fgc
    gts = tot_f
    j_end = 1.0 - (gts - f_b) / jnp.maximum(gts + n_b - f_b, 1.0)
    j_sta = 1.0 - (gts - (f_b - fgc)) / jnp.maximum(
        gts + (n_b - cnt) - (f_b - fgc), 1.0)
    eb = (lax.broadcasted_iota(jnp.int32, (NCLS, NB), 1).astype(jnp.float32)
          + 0.5) / NB
    term = jnp.sum(eb * (j_end - j_sta), axis=1, keepdims=True)
    lv = jnp.sum(jnp.where(gts > 0, term, 0.0)) / NCLS
    ce = jnp.sum(ce_ref[0]) / jnp.sum(ce_ref[1])
    out_ref[...] = jnp.full((8, 128), CE_W * ce + LV_W * lv, jnp.float32)


def _sc_hist(idx_hbm, zeros_hbm, out_hbm, buf0, buf1, hist_v, sem0, sem1):
    cid = lax.axis_index("c")
    sid = lax.axis_index("s")
    wid = sid * NSC + cid
    base = wid * PER_W

    pltpu.sync_copy(zeros_hbm, hist_v)

    bufs = (buf0, buf1)
    sems = (sem0, sem1)

    def load(ch, slot):
        return pltpu.make_async_copy(
            idx_hbm.at[pl.ds(base + ch * CHUNK, CHUNK)], bufs[slot], sems[slot])

    ones = jnp.ones((16,), jnp.float32)

    def make_group_body(buf):
        def group_body(g, carry):
            off = g * 2048
            for t in range(128):
                vec = buf[pl.ds(off + t * 16, 16)]
                plsc.addupdate_scatter(hist_v, [vec], ones)
            return carry
        return group_body

    load(0, 0).start()
    for ch in range(NCHUNK):
        slot = ch % 2
        if ch + 1 < NCHUNK:
            load(ch + 1, 1 - slot).start()
        load(ch, slot).wait()
        lax.fori_loop(0, GROUPS, make_group_body(bufs[slot]), 0)

    pltpu.sync_copy(hist_v, out_hbm.at[wid])


def kernel(inputs, targets):
    targets = targets.astype(jnp.int32)

    idx, ce_parts = pl.pallas_call(
        _p1_body,
        grid=(BATCH, GRID_H),
        in_specs=[
            pl.BlockSpec((1, C, HBLK, WDIM), lambda b, h: (b, 0, h, 0)),
            pl.BlockSpec((1, HBLK, WDIM), lambda b, h: (b, h, 0)),
        ],
        out_specs=[
            pl.BlockSpec((NCLS, 1, HBLK, WDIM), lambda b, h: (0, b, h, 0)),
            pl.BlockSpec((2, 8, 128), lambda b, h: (0, 0, 0)),
        ],
        out_shape=[
            jax.ShapeDtypeStruct((NCLS, BATCH, HDIM, WDIM), jnp.int32),
            jax.ShapeDtypeStruct((2, 8, 128), jnp.float32),
        ],
    )(inputs, targets)

    idx1d = idx.reshape(NIDX)
    zeros = jnp.zeros((HSIZE,), jnp.float32)

    sc_fn = pl.kernel(
        _sc_hist,
        out_type=jax.ShapeDtypeStruct((NW, HSIZE), jnp.float32),
        mesh=plsc.VectorSubcoreMesh(core_axis_name="c", subcore_axis_name="s"),
        compiler_params=pltpu.CompilerParams(needs_layout_passes=False),
        scratch_types=[
            pltpu.VMEM((CHUNK,), jnp.int32),
            pltpu.VMEM((CHUNK,), jnp.int32),
            pltpu.VMEM((HSIZE,), jnp.float32),
            pltpu.SemaphoreType.DMA,
            pltpu.SemaphoreType.DMA,
        ],
    )
    hist = sc_fn(idx1d, zeros)

    out = pl.pallas_call(
        _p3_body,
        in_specs=[
            pl.BlockSpec((NW * C, NB), lambda: (0, 0)),
            pl.BlockSpec((2, 8, 128), lambda: (0, 0, 0)),
        ],
        out_specs=pl.BlockSpec((8, 128), lambda: (0, 0)),
        out_shape=jax.ShapeDtypeStruct((8, 128), jnp.float32),
    )(hist.reshape(NW * C, NB), ce_parts)

    return out[0, 0]


# speedup vs baseline: 72.8219x; 1.4828x over previous
import jax
import jax.numpy as jnp
from jax import lax
from jax.experimental import pallas as pl
from jax.experimental.pallas import tpu as pltpu
from jax.experimental.pallas import tpu_sc as plsc

IGNORE = 19
CE_W = 0.5
LV_W = 0.5
NCLS = 10
C = 20
NB = 2048
HSIZE = 2 * NCLS * NB

BATCH, HDIM, WDIM = 4, 512, 512
NPIX = BATCH * HDIM * WDIM
HBLK = 32
GRID_H = HDIM // HBLK
ROWS_STEP = HBLK * WDIM // 128

NIDX = NCLS * NPIX
NSC, NTEC = 2, 16
NW = NSC * NTEC
PER_W = NIDX // NW
CHUNK = 32768
NCHUNK = PER_W // CHUNK
GROUPS = CHUNK // 2048


def _p1_body(x_ref, t_ref, idx_ref, ce_ref):
    first = jnp.logical_and(pl.program_id(0) == 0, pl.program_id(1) == 0)

    @pl.when(first)
    def _():
        ce_ref[...] = jnp.zeros_like(ce_ref)

    x = x_ref[0]
    t = t_ref[0]
    m = jnp.max(x, axis=0)
    ex = jnp.exp(x - m[None])
    s = jnp.sum(ex, axis=0)
    lse = jnp.log(s) + m
    cls = lax.broadcasted_iota(jnp.int32, (C, HBLK, WDIM), 0)
    xt = jnp.sum(jnp.where(t[None] == cls, x, 0.0), axis=0)
    maskf = (t != IGNORE).astype(jnp.float32)
    nll = (lse - xt) * maskf
    ce_ref[0] += jnp.sum(nll.reshape(HBLK, WDIM // 128, 128), axis=1)
    ce_ref[1] += jnp.sum(maskf.reshape(HBLK, WDIM // 128, 128), axis=1)

    s_inv = 1.0 / s
    for c in range(NCLS):
        p = ex[c] * s_inv
        fg = t == c
        e = jnp.abs(fg.astype(jnp.float32) - p) * maskf
        b = jnp.minimum((e * NB).astype(jnp.int32), NB - 1)
        idx = c * NB + b + jnp.where(fg, NCLS * NB, 0)
        idx_ref[c] = idx.reshape(ROWS_STEP, 128)


def _p3_body(hist_ref, ce_ref, out_ref):
    tot = hist_ref[pl.ds(0, C)]
    for w in range(1, NW):
        tot += hist_ref[pl.ds(w * C, C)]
    fgc = tot[NCLS:]
    cnt = tot[:NCLS] + fgc

    def cum(v):
        sh = 1
        while sh < NB:
            z = jnp.zeros((NCLS, sh), jnp.float32)
            v = v + jnp.concatenate([z, v[:, :-sh]], axis=1)
            sh *= 2
        return v

    cum_c = cum(cnt)
    cum_f = cum(fgc)
    tot_c = cum_c[:, -1:]
    tot_f = cum_f[:, -1:]
    n_b = tot_c - cum_c + cnt
    f_b = tot_f - cum_f + fgc
    gts = tot_f
    j_end = 1.0 - (gts - f_b) / jnp.maximum(gts + n_b - f_b, 1.0)
    j_sta = 1.0 - (gts - (f_b - fgc)) / jnp.maximum(
        gts + (n_b - cnt) - (f_b - fgc), 1.0)
    eb = (lax.broadcasted_iota(jnp.int32, (NCLS, NB), 1).astype(jnp.float32)
          + 0.5) / NB
    term = jnp.sum(eb * (j_end - j_sta), axis=1, keepdims=True)
    lv = jnp.sum(jnp.where(gts > 0, term, 0.0)) / NCLS
    ce = jnp.sum(ce_ref[0]) / jnp.sum(ce_ref[1])
    out_ref[...] = jnp.full((8, 128), CE_W * ce + LV_W * lv, jnp.float32)


def _sc_hist(idx_hbm, zeros_hbm, out_hbm, buf0, buf1, hist_v, sem0, sem1):
    cid = lax.axis_index("c")
    sid = lax.axis_index("s")
    wid = sid * NSC + cid
    base = wid * PER_W

    pltpu.sync_copy(zeros_hbm, hist_v)

    bufs = (buf0, buf1)
    sems = (sem0, sem1)

    def load(ch, slot):
        return pltpu.make_async_copy(
            idx_hbm.at[pl.ds(base + ch * CHUNK, CHUNK)], bufs[slot], sems[slot])

    ones = jnp.ones((16,), jnp.float32)

    def make_group_body(buf):
        def group_body(g, carry):
            off = g * 2048
            for t in range(128):
                vec = buf[pl.ds(off + t * 16, 16)]
                plsc.addupdate_scatter(hist_v, [vec], ones)
            return carry
        return group_body

    load(0, 0).start()
    for ch in range(NCHUNK):
        slot = ch % 2
        if ch + 1 < NCHUNK:
            load(ch + 1, 1 - slot).start()
        load(ch, slot).wait()
        lax.fori_loop(0, GROUPS, make_group_body(bufs[slot]), 0)

    pltpu.sync_copy(hist_v, out_hbm.at[wid])


def kernel(inputs, targets):
    targets = targets.astype(jnp.int32)

    idx, ce_parts = pl.pallas_call(
        _p1_body,
        grid=(BATCH, GRID_H),
        in_specs=[
            pl.BlockSpec((1, C, HBLK, WDIM), lambda b, h: (b, 0, h, 0)),
            pl.BlockSpec((1, HBLK, WDIM), lambda b, h: (b, h, 0)),
        ],
        out_specs=[
            pl.BlockSpec((NCLS, ROWS_STEP, 128),
                         lambda b, h: (0, b * GRID_H + h, 0)),
            pl.BlockSpec((2, HBLK, 128), lambda b, h: (0, 0, 0)),
        ],
        out_shape=[
            jax.ShapeDtypeStruct((NCLS, NPIX // 128, 128), jnp.int32),
            jax.ShapeDtypeStruct((2, HBLK, 128), jnp.float32),
        ],
    )(inputs, targets)

    idx1d = idx.reshape(NIDX)
    zeros = jnp.zeros((HSIZE,), jnp.float32)

    sc_fn = pl.kernel(
        _sc_hist,
        out_type=jax.ShapeDtypeStruct((NW, HSIZE), jnp.float32),
        mesh=plsc.VectorSubcoreMesh(core_axis_name="c", subcore_axis_name="s"),
        compiler_params=pltpu.CompilerParams(needs_layout_passes=False),
        scratch_types=[
            pltpu.VMEM((CHUNK,), jnp.int32),
            pltpu.VMEM((CHUNK,), jnp.int32),
            pltpu.VMEM((HSIZE,), jnp.float32),
            pltpu.SemaphoreType.DMA,
            pltpu.SemaphoreType.DMA,
        ],
    )
    hist = sc_fn(idx1d, zeros)

    out = pl.pallas_call(
        _p3_body,
        in_specs=[
            pl.BlockSpec((NW * C, NB), lambda: (0, 0)),
            pl.BlockSpec((2, HBLK, 128), lambda: (0, 0, 0)),
        ],
        out_specs=pl.BlockSpec((8, 128), lambda: (0, 0)),
        out_shape=jax.ShapeDtypeStruct((8, 128), jnp.float32),
    )(hist.reshape(NW * C, NB), ce_parts)

    return out[0, 0]


# speedup vs baseline: 110.2114x; 1.5134x over previous
import jax
import jax.numpy as jnp
from jax import lax
from jax.experimental import pallas as pl
from jax.experimental.pallas import tpu as pltpu
from jax.experimental.pallas import tpu_sc as plsc

IGNORE = 19
CE_W = 0.5
LV_W = 0.5
NCLS = 10
C = 20
NB = 2048
HSIZE = 2 * NCLS * NB

BATCH, HDIM, WDIM = 4, 512, 512
NPIX = BATCH * HDIM * WDIM
HBLK = 32
GRID_H = HDIM // HBLK
ROWS_STEP = HBLK * WDIM // 128

NIDX = NCLS * NPIX
NSC, NTEC = 2, 16
NW = NSC * NTEC
PER_W = NIDX // NW
CHUNK = 32768
NCHUNK = PER_W // CHUNK
GROUPS = CHUNK // 2048


def _p1_body(x_ref, t_ref, idx_ref, ce_ref):
    first = jnp.logical_and(pl.program_id(0) == 0, pl.program_id(1) == 0)

    @pl.when(first)
    def _():
        ce_ref[...] = jnp.zeros_like(ce_ref)

    x = x_ref[0]
    t = t_ref[0]
    m = jnp.max(x, axis=0)
    ex = jnp.exp(x - m[None])
    s = jnp.sum(ex, axis=0)
    lse = jnp.log(s) + m
    cls = lax.broadcasted_iota(jnp.int32, (C, HBLK, WDIM), 0)
    xt = jnp.sum(jnp.where(t[None] == cls, x, 0.0), axis=0)
    maskf = (t != IGNORE).astype(jnp.float32)
    nll = (lse - xt) * maskf
    ce_ref[0] += jnp.sum(nll.reshape(HBLK, WDIM // 128, 128), axis=1)
    ce_ref[1] += jnp.sum(maskf.reshape(HBLK, WDIM // 128, 128), axis=1)

    s_inv = 1.0 / s
    for c in range(NCLS):
        p = ex[c] * s_inv
        fg = t == c
        e = jnp.abs(fg.astype(jnp.float32) - p) * maskf
        b = jnp.minimum((e * NB).astype(jnp.int32), NB - 1)
        idx = c * NB + b + jnp.where(fg, NCLS * NB, 0)
        idx_ref[c] = idx.reshape(ROWS_STEP, 128)


def _p3_body(hist_ref, ce_ref, out_ref):
    tot = hist_ref[pl.ds(0, C)]
    for w in range(1, NW):
        tot += hist_ref[pl.ds(w * C, C)]
    fgc = tot[NCLS:]
    cnt = tot[:NCLS] + fgc

    def cum(v):
        sh = 1
        while sh < NB:
            z = jnp.zeros((NCLS, sh), jnp.float32)
            v = v + jnp.concatenate([z, v[:, :-sh]], axis=1)
            sh *= 2
        return v

    cum_c = cum(cnt)
    cum_f = cum(fgc)
    tot_c = cum_c[:, -1:]
    tot_f = cum_f[:, -1:]
    n_b = tot_c - cum_c + cnt
    f_b = tot_f - cum_f + fgc
    gts = tot_f
    j_end = 1.0 - (gts - f_b) / jnp.maximum(gts + n_b - f_b, 1.0)
    j_sta = 1.0 - (gts - (f_b - fgc)) / jnp.maximum(
        gts + (n_b - cnt) - (f_b - fgc), 1.0)
    eb = (lax.broadcasted_iota(jnp.int32, (NCLS, NB), 1).astype(jnp.float32)
          + 0.5) / NB
    term = jnp.sum(eb * (j_end - j_sta), axis=1, keepdims=True)
    lv = jnp.sum(jnp.where(gts > 0, term, 0.0)) / NCLS
    ce = jnp.sum(ce_ref[0]) / jnp.sum(ce_ref[1])
    out_ref[...] = jnp.full((8, 128), CE_W * ce + LV_W * lv, jnp.float32)


def _sc_hist(idx_hbm, zeros_hbm, out_hbm, buf0, buf1, hist_v, sem0, sem1):
    cid = lax.axis_index("c")
    sid = lax.axis_index("s")
    wid = sid * NSC + cid
    base = wid * PER_W

    pltpu.sync_copy(zeros_hbm, hist_v)

    bufs = (buf0, buf1)
    sems = (sem0, sem1)

    def load(ch, slot):
        return pltpu.make_async_copy(
            idx_hbm.at[pl.ds(base + ch * CHUNK, CHUNK)], bufs[slot], sems[slot])

    ones = jnp.ones((16,), jnp.float32)

    def scatter_chunk(buf):
        @plsc.parallel_loop(0, CHUNK // 16, unroll=8)
        def _(t):
            vec = buf[pl.ds(t * 16, 16)]
            plsc.addupdate_scatter(hist_v, [vec], ones)

    load(0, 0).start()
    for ch in range(NCHUNK):
        slot = ch % 2
        if ch + 1 < NCHUNK:
            load(ch + 1, 1 - slot).start()
        load(ch, slot).wait()
        scatter_chunk(bufs[slot])

    pltpu.sync_copy(hist_v, out_hbm.at[wid])


def kernel(inputs, targets):
    targets = targets.astype(jnp.int32)

    idx, ce_parts = pl.pallas_call(
        _p1_body,
        grid=(BATCH, GRID_H),
        in_specs=[
            pl.BlockSpec((1, C, HBLK, WDIM), lambda b, h: (b, 0, h, 0)),
            pl.BlockSpec((1, HBLK, WDIM), lambda b, h: (b, h, 0)),
        ],
        out_specs=[
            pl.BlockSpec((NCLS, ROWS_STEP, 128),
                         lambda b, h: (0, b * GRID_H + h, 0)),
            pl.BlockSpec((2, HBLK, 128), lambda b, h: (0, 0, 0)),
        ],
        out_shape=[
            jax.ShapeDtypeStruct((NCLS, NPIX // 128, 128), jnp.int32),
            jax.ShapeDtypeStruct((2, HBLK, 128), jnp.float32),
        ],
    )(inputs, targets)

    idx1d = idx.reshape(NIDX)
    zeros = jnp.zeros((HSIZE,), jnp.float32)

    sc_fn = pl.kernel(
        _sc_hist,
        out_type=jax.ShapeDtypeStruct((NW, HSIZE), jnp.float32),
        mesh=plsc.VectorSubcoreMesh(core_axis_name="c", subcore_axis_name="s"),
        compiler_params=pltpu.CompilerParams(needs_layout_passes=False),
        scratch_types=[
            pltpu.VMEM((CHUNK,), jnp.int32),
            pltpu.VMEM((CHUNK,), jnp.int32),
            pltpu.VMEM((HSIZE,), jnp.float32),
            pltpu.SemaphoreType.DMA,
            pltpu.SemaphoreType.DMA,
        ],
    )
    hist = sc_fn(idx1d, zeros)

    out = pl.pallas_call(
        _p3_body,
        in_specs=[
            pl.BlockSpec((NW * C, NB), lambda: (0, 0)),
            pl.BlockSpec((2, HBLK, 128), lambda: (0, 0, 0)),
        ],
        out_specs=pl.BlockSpec((8, 128), lambda: (0, 0)),
        out_shape=jax.ShapeDtypeStruct((8, 128), jnp.float32),
    )(hist.reshape(NW * C, NB), ce_parts)

    return out[0, 0]


# speedup vs baseline: 123.3419x; 1.1191x over previous
import jax
import jax.numpy as jnp
from jax import lax
from jax.experimental import pallas as pl
from jax.experimental.pallas import tpu as pltpu
from jax.experimental.pallas import tpu_sc as plsc

IGNORE = 19
CE_W = 0.5
LV_W = 0.5
NCLS = 10
C = 20
NB = 2048
HSIZE = 2 * NCLS * NB

BATCH, HDIM, WDIM = 4, 512, 512
NPIX = BATCH * HDIM * WDIM
HBLK = 64
GRID_H = HDIM // HBLK
ROWS_STEP = HBLK * WDIM // 128

NIDX = NCLS * NPIX
NSC, NTEC = 2, 16
NW = NSC * NTEC
PER_W = NIDX // NW
CHUNK = 32768
NCHUNK = PER_W // CHUNK
GROUPS = CHUNK // 2048


def _p1_body(x_ref, t_ref, idx_ref, ce_ref):
    first = jnp.logical_and(pl.program_id(0) == 0, pl.program_id(1) == 0)

    @pl.when(first)
    def _():
        ce_ref[...] = jnp.zeros_like(ce_ref)

    x = x_ref[0]
    t = t_ref[0]
    m = jnp.max(x, axis=0)
    ex = jnp.exp(x - m[None])
    s = jnp.sum(ex, axis=0)
    lse = jnp.log(s) + m
    xt = jnp.zeros((HBLK, WDIM), jnp.float32)
    fgf = []
    for c in range(C):
        cmp = t == c
        xt = xt + jnp.where(cmp, x[c], 0.0)
        if c < NCLS:
            fgf.append(cmp.astype(jnp.float32))
    maskf = (t != IGNORE).astype(jnp.float32)
    nll = (lse - xt) * maskf
    ce_ref[0] += jnp.sum(nll.reshape(HBLK, WDIM // 128, 128), axis=1)
    ce_ref[1] += jnp.sum(maskf.reshape(HBLK, WDIM // 128, 128), axis=1)

    s_inv = 1.0 / s
    for c in range(NCLS):
        p = ex[c] * s_inv
        e = jnp.abs(fgf[c] - p) * maskf
        idx_f = jnp.minimum(e * NB, NB - 1.0) + fgf[c] * (NCLS * NB) + c * NB
        idx_ref[c] = idx_f.astype(jnp.int32).reshape(ROWS_STEP, 128)


def _p3_body(hist_ref, ce_ref, out_ref):
    tot = hist_ref[pl.ds(0, C)]
    for w in range(1, NW):
        tot += hist_ref[pl.ds(w * C, C)]
    fgc = tot[NCLS:]
    cnt = tot[:NCLS] + fgc

    def cum(v):
        sh = 1
        while sh < NB:
            z = jnp.zeros((NCLS, sh), jnp.float32)
            v = v + jnp.concatenate([z, v[:, :-sh]], axis=1)
            sh *= 2
        return v

    cum_c = cum(cnt)
    cum_f = cum(fgc)
    tot_c = cum_c[:, -1:]
    tot_f = cum_f[:, -1:]
    n_b = tot_c - cum_c + cnt
    f_b = tot_f - cum_f + fgc
    gts = tot_f
    j_end = 1.0 - (gts - f_b) / jnp.maximum(gts + n_b - f_b, 1.0)
    j_sta = 1.0 - (gts - (f_b - fgc)) / jnp.maximum(
        gts + (n_b - cnt) - (f_b - fgc), 1.0)
    eb = (lax.broadcasted_iota(jnp.int32, (NCLS, NB), 1).astype(jnp.float32)
          + 0.5) / NB
    term = jnp.sum(eb * (j_end - j_sta), axis=1, keepdims=True)
    lv = jnp.sum(jnp.where(gts > 0, term, 0.0)) / NCLS
    ce = jnp.sum(ce_ref[0]) / jnp.sum(ce_ref[1])
    out_ref[...] = jnp.full((8, 128), CE_W * ce + LV_W * lv, jnp.float32)


def _sc_hist(idx_hbm, zeros_hbm, out_hbm, buf0, buf1, hist_v, sem0, sem1):
    cid = lax.axis_index("c")
    sid = lax.axis_index("s")
    wid = sid * NSC + cid
    base = wid * PER_W

    pltpu.sync_copy(zeros_hbm, hist_v)

    bufs = (buf0, buf1)
    sems = (sem0, sem1)

    def load(ch, slot):
        return pltpu.make_async_copy(
            idx_hbm.at[pl.ds(base + ch * CHUNK, CHUNK)], bufs[slot], sems[slot])

    ones = jnp.ones((16,), jnp.float32)

    def scatter_chunk(buf):
        @plsc.parallel_loop(0, CHUNK // 16, unroll=8)
        def _(t):
            vec = buf[pl.ds(t * 16, 16)]
            plsc.addupdate_scatter(hist_v, [vec], ones)

    load(0, 0).start()
    for ch in range(NCHUNK):
        slot = ch % 2
        if ch + 1 < NCHUNK:
            load(ch + 1, 1 - slot).start()
        load(ch, slot).wait()
        scatter_chunk(bufs[slot])

    pltpu.sync_copy(hist_v, out_hbm.at[wid])


def kernel(inputs, targets):
    targets = targets.astype(jnp.int32)

    idx, ce_parts = pl.pallas_call(
        _p1_body,
        grid=(BATCH, GRID_H),
        in_specs=[
            pl.BlockSpec((1, C, HBLK, WDIM), lambda b, h: (b, 0, h, 0)),
            pl.BlockSpec((1, HBLK, WDIM), lambda b, h: (b, h, 0)),
        ],
        out_specs=[
            pl.BlockSpec((NCLS, ROWS_STEP, 128),
                         lambda b, h: (0, b * GRID_H + h, 0)),
            pl.BlockSpec((2, HBLK, 128), lambda b, h: (0, 0, 0)),
        ],
        out_shape=[
            jax.ShapeDtypeStruct((NCLS, NPIX // 128, 128), jnp.int32),
            jax.ShapeDtypeStruct((2, HBLK, 128), jnp.float32),
        ],
    )(inputs, targets)

    idx1d = idx.reshape(NIDX)
    zeros = jnp.zeros((HSIZE,), jnp.float32)

    sc_fn = pl.kernel(
        _sc_hist,
        out_type=jax.ShapeDtypeStruct((NW, HSIZE), jnp.float32),
        mesh=plsc.VectorSubcoreMesh(core_axis_name="c", subcore_axis_name="s"),
        compiler_params=pltpu.CompilerParams(needs_layout_passes=False),
        scratch_types=[
            pltpu.VMEM((CHUNK,), jnp.int32),
            pltpu.VMEM((CHUNK,), jnp.int32),
            pltpu.VMEM((HSIZE,), jnp.float32),
            pltpu.SemaphoreType.DMA,
            pltpu.SemaphoreType.DMA,
        ],
    )
    hist = sc_fn(idx1d, zeros)

    out = pl.pallas_call(
        _p3_body,
        in_specs=[
            pl.BlockSpec((NW * C, NB), lambda: (0, 0)),
            pl.BlockSpec((2, HBLK, 128), lambda: (0, 0, 0)),
        ],
        out_specs=pl.BlockSpec((8, 128), lambda: (0, 0)),
        out_shape=jax.ShapeDtypeStruct((8, 128), jnp.float32),
    )(hist.reshape(NW * C, NB), ce_parts)

    return out[0, 0]
